# baseline (speedup 1.0000x reference)
import functools

import jax
import jax.numpy as jnp
from jax import lax
from jax.experimental import pallas as pl
from jax.experimental.pallas import tpu as pltpu
from jax.experimental.pallas import tpu_sc as plsc

_VOCAB = 1000000
_DIM = 64
_BATCH = 16384
_NNEG = 81920

_NC, _NS = 2, 16
_NW = _NC * _NS
_CHUNK = 128
_GRP = 16
_P_PER = _BATCH // _NW
_N_PER = _NNEG // _NW
_P_CH = _P_PER // _CHUNK
_N_CH = _N_PER // _CHUNK


def _sc_body(in_emb, out_emb, cw, ctx, negw, negc,
             ce_out, ps_out, ns_out,
             idx_pa, idx_pb, idx_na, idx_nb,
             a_buf, b_buf, ps_buf, ns_buf, sem_a, sem_b):
  wid = lax.axis_index("s") * _NC + lax.axis_index("c")
  lanes = lax.iota(jnp.int32, _GRP)

  pltpu.sync_copy(cw.at[wid], idx_pa)
  pltpu.sync_copy(ctx.at[wid], idx_pb)
  pltpu.sync_copy(negc.at[wid], idx_na)
  pltpu.sync_copy(negw.at[wid], idx_nb)

  def dot_groups(score_buf, score_base):
    def group(g, _):
      rows = g * _GRP + lanes

      def dstep(d, acc):
        cols = jnp.full((_GRP,), d, jnp.int32)
        va = plsc.load_gather(a_buf, [rows, cols])
        vb = plsc.load_gather(b_buf, [rows, cols])
        return acc + va * vb

      acc = lax.fori_loop(0, _DIM, dstep, jnp.zeros((_GRP,), jnp.float32),
                          unroll=8)
      score_buf[pl.ds(score_base + g * _GRP, _GRP)] = acc
      return 0

    lax.fori_loop(0, _CHUNK // _GRP, group, 0)

  def pos_chunk(c, _):
    ca = pltpu.async_copy(in_emb.at[idx_pa.at[c]], a_buf, sem_a)
    cb = pltpu.async_copy(out_emb.at[idx_pb.at[c]], b_buf, sem_b)
    ca.wait()
    cb.wait()
    pltpu.sync_copy(a_buf, ce_out.at[pl.ds(wid * _P_PER + c * _CHUNK, _CHUNK)])
    dot_groups(ps_buf, c * _CHUNK)
    return 0

  lax.fori_loop(0, _P_CH, pos_chunk, 0)
  pltpu.sync_copy(ps_buf, ps_out.at[pl.ds(wid * _P_PER, _P_PER)])

  def neg_chunk(c, _):
    ca = pltpu.async_copy(in_emb.at[idx_na.at[c]], a_buf, sem_a)
    cb = pltpu.async_copy(out_emb.at[idx_nb.at[c]], b_buf, sem_b)
    ca.wait()
    cb.wait()
    dot_groups(ns_buf, c * _CHUNK)
    return 0

  lax.fori_loop(0, _N_CH, neg_chunk, 0)
  pltpu.sync_copy(ns_buf, ns_out.at[pl.ds(wid * _N_PER, _N_PER)])


_sc_call = functools.partial(
    pl.kernel,
    out_type=(
        jax.ShapeDtypeStruct((_BATCH, _DIM), jnp.float32),
        jax.ShapeDtypeStruct((_BATCH,), jnp.float32),
        jax.ShapeDtypeStruct((_NNEG,), jnp.float32),
    ),
    mesh=plsc.VectorSubcoreMesh(core_axis_name="c", subcore_axis_name="s",
                                num_cores=_NC, num_subcores=_NS),
    compiler_params=pltpu.CompilerParams(needs_layout_passes=False,
                                         use_tc_tiling_on_sc=False),
    scratch_types=(
        pltpu.VMEM((_P_CH, _CHUNK), jnp.int32),
        pltpu.VMEM((_P_CH, _CHUNK), jnp.int32),
        pltpu.VMEM((_N_CH, _CHUNK), jnp.int32),
        pltpu.VMEM((_N_CH, _CHUNK), jnp.int32),
        pltpu.VMEM((_CHUNK, _DIM), jnp.float32),
        pltpu.VMEM((_CHUNK, _DIM), jnp.float32),
        pltpu.VMEM((_P_PER,), jnp.float32),
        pltpu.VMEM((_N_PER,), jnp.float32),
        pltpu.SemaphoreType.DMA,
        pltpu.SemaphoreType.DMA,
    ),
)(_sc_body)


def _log_sigmoid(x):
  return jnp.minimum(x, 0.0) - jnp.log1p(jnp.exp(-jnp.abs(x)))


def _loss_body(ps_ref, ns_ref, out_ref):
  pos_mean = jnp.mean(_log_sigmoid(ps_ref[...]))
  neg_mean = jnp.mean(_log_sigmoid(-ns_ref[...]))
  out_ref[0] = pos_mean
  out_ref[1] = neg_mean
  out_ref[2] = -(pos_mean + neg_mean)


_loss_call = pl.pallas_call(
    _loss_body,
    out_shape=jax.ShapeDtypeStruct((3,), jnp.float32),
    out_specs=pl.BlockSpec(memory_space=pltpu.SMEM),
)


def kernel(center_words, context_words, negative_words, negative_centers,
           in_emb, out_emb):
  cw = center_words.astype(jnp.int32).reshape(_NW, _P_CH, _CHUNK)
  ctx = context_words.astype(jnp.int32).reshape(_NW, _P_CH, _CHUNK)
  negw = negative_words.astype(jnp.int32).reshape(_NW, _N_CH, _CHUNK)
  negc = negative_centers.astype(jnp.int32).reshape(_NW, _N_CH, _CHUNK)
  ce, ps, ns = _sc_call(in_emb, out_emb, cw, ctx, negw, negc)
  losses = _loss_call(ps.reshape(_BATCH // 128, 128),
                      ns.reshape(_NNEG // 128, 128))
  return (losses[0], ps, ce, losses[1], ns, losses[2])
